# Initial kernel scaffold; baseline (speedup 1.0000x reference)
import jax
import jax.numpy as jnp
from jax.experimental import pallas as pl


def kernel(x):
    raise NotImplementedError("write your pallas kernel here")


# speedup vs baseline: 19.9364x; 19.9364x over previous
import functools

import jax
import jax.numpy as jnp
from jax import lax
from jax.experimental import pallas as pl
from jax.experimental.pallas import tpu as pltpu

_MAX_OBJ = 100
_IOU_THRES = 0.45
_SCORE_THRES = 0.25
_MAX_WH = 640.0
_NEG_INF = float("-inf")
_BIG_I32 = 2**30


def _nms_kernel(x_ref, out_ref, scr_s, nx1_s, ny1_s, nx2_s, ny2_s, a2_s,
                bx1_s, by1_s, bx2_s, by2_s, cat_s, *, B, R, NCLS):
    b = pl.program_id(0)

    cx = x_ref[0, 0]
    cy = x_ref[0, 1]
    w = x_ref[0, 2]
    h = x_ref[0, 3]
    conf = x_ref[0, 4]

    cxr = cx.astype(jnp.bfloat16).astype(jnp.float32)
    cyr = cy.astype(jnp.bfloat16).astype(jnp.float32)
    wr = w.astype(jnp.bfloat16).astype(jnp.float32)
    hr = h.astype(jnp.bfloat16).astype(jnp.float32)
    bx1 = cxr - 0.5 * wr
    by1 = cyr - 0.5 * hr
    bx2 = cxr + 0.5 * wr
    by2 = cyr + 0.5 * hr

    sc0 = x_ref[0, 5] * conf

    def cls_body(c, carry):
        msc, cat = carry
        sc = x_ref[0, 5 + c] * conf
        cat = jnp.where(sc > msc, c, cat)
        msc = jnp.maximum(msc, sc)
        return msc, cat

    msc, cat = lax.fori_loop(1, NCLS, cls_body,
                             (sc0, jnp.zeros_like(sc0, jnp.int32)))
    catf = cat.astype(jnp.float32)

    nx1 = bx1 + catf * _MAX_WH
    ny1 = by1 + catf * _MAX_WH
    nx2 = bx2 + catf * _MAX_WH
    ny2 = by2 + catf * _MAX_WH
    a2 = (nx2 - nx1) * (ny2 - ny1)
    scr0 = jnp.where(msc > _SCORE_THRES, msc, _NEG_INF)

    scr_s[b] = scr0
    nx1_s[b] = nx1
    ny1_s[b] = ny1
    nx2_s[b] = nx2
    ny2_s[b] = ny2
    a2_s[b] = a2
    bx1_s[b] = bx1
    by1_s[b] = by1
    bx2_s[b] = bx2
    by2_s[b] = by2
    cat_s[b] = catf

    @pl.when(b == B - 1)
    def _run_nms():
        lin = (lax.broadcasted_iota(jnp.int32, (1, R, 128), 1) * 128
               + lax.broadcasted_iota(jnp.int32, (1, R, 128), 2))
        bid = lax.broadcasted_iota(jnp.int32, (B, 1, 1), 0).astype(jnp.float32)
        lane = lax.broadcasted_iota(jnp.int32, (1, 1, 128), 2)

        def ext(ref, oh):
            v = jnp.where(oh, ref[...], 0.0)
            return jnp.sum(jnp.sum(v, axis=1, keepdims=True),
                           axis=2, keepdims=True)

        def body(i, _):
            scr = scr_s[...]
            m = jnp.max(jnp.max(scr, axis=1, keepdims=True),
                        axis=2, keepdims=True)
            ok = m > _NEG_INF
            ml = jnp.where(scr == m, lin, _BIG_I32)
            j = jnp.min(jnp.min(ml, axis=1, keepdims=True),
                        axis=2, keepdims=True)
            oh = lin == j

            bx1j = ext(bx1_s, oh)
            by1j = ext(by1_s, oh)
            bx2j = ext(bx2_s, oh)
            by2j = ext(by2_s, oh)
            catj = ext(cat_s, oh)

            off = catj * _MAX_WH
            nx1j = bx1j + off
            ny1j = by1j + off
            nx2j = bx2j + off
            ny2j = by2j + off
            a1 = (nx2j - nx1j) * (ny2j - ny1j)

            xx1 = jnp.maximum(nx1j, nx1_s[...])
            yy1 = jnp.maximum(ny1j, ny1_s[...])
            xx2 = jnp.minimum(nx2j, nx2_s[...])
            yy2 = jnp.minimum(ny2j, ny2_s[...])
            inter = (jnp.maximum(xx2 - xx1, 0.0)
                     * jnp.maximum(yy2 - yy1, 0.0))
            iou = inter / (a1 + a2_s[...] - inter + 1e-9)
            scr_s[...] = jnp.where((iou > _IOU_THRES) | oh, _NEG_INF, scr)

            okf = ok.astype(jnp.float32)
            scorej = jnp.where(ok, m, 0.0)
            row = (jnp.where(lane == 0, bid, 0.0)
                   + jnp.where(lane == 1, bx1j, 0.0)
                   + jnp.where(lane == 2, by1j, 0.0)
                   + jnp.where(lane == 3, bx2j, 0.0)
                   + jnp.where(lane == 4, by2j, 0.0)
                   + jnp.where(lane == 5, catj, 0.0)
                   + jnp.where(lane == 6, scorej, 0.0)) * okf
            out_ref[:, pl.ds(i, 1), :] = row
            return 0

        lax.fori_loop(0, _MAX_OBJ, body, 0)


@jax.jit
def kernel(x):
    B, N, C = x.shape
    R = (N + 127) // 128
    NP = R * 128
    NCLS = C - 5

    xp = jnp.pad(x, ((0, 0), (0, NP - N), (0, 0)))
    xt = xp.transpose(0, 2, 1).reshape(B, C, R, 128)

    slab = pltpu.VMEM((B, R, 128), jnp.float32)
    out = pl.pallas_call(
        functools.partial(_nms_kernel, B=B, R=R, NCLS=NCLS),
        grid=(B,),
        in_specs=[pl.BlockSpec((1, C, R, 128), lambda b: (b, 0, 0, 0))],
        out_specs=pl.BlockSpec((B, _MAX_OBJ, 128), lambda b: (0, 0, 0)),
        out_shape=jax.ShapeDtypeStruct((B, _MAX_OBJ, 128), jnp.float32),
        scratch_shapes=[slab] * 11,
        compiler_params=pltpu.CompilerParams(
            dimension_semantics=("arbitrary",)),
    )(xt)
    return out[:, :, :7].reshape(B * _MAX_OBJ, 7)
